# Initial kernel scaffold; baseline (speedup 1.0000x reference)
import jax
import jax.numpy as jnp
from jax.experimental import pallas as pl


def kernel(coords, features, W1, b1, W2, b2, W3, b3):
    raise NotImplementedError("write your pallas kernel here")


# speedup vs baseline: 2.0619x; 2.0619x over previous
import jax
import jax.numpy as jnp
from jax.experimental import pallas as pl
from jax.experimental.pallas import tpu as pltpu

N = 10000
NPAD = 10240
NROWS = NPAD // 128
K = 50
KPAD = 64
R2 = 0.0625
L1, L2, L3 = 32, 32, 64
RB = 1024
NRB = NPAD // RB
PADVAL = 1.0e6


def _fps_body(xs_ref, ys_ref, zs_ref, idx_ref, cx_ref, cy_ref, cz_ref):
    xs = xs_ref[...]
    ys = ys_ref[...]
    zs = zs_ref[...]
    rowi = jax.lax.broadcasted_iota(jnp.int32, (NROWS, 128), 0)
    coli = jax.lax.broadcasted_iota(jnp.int32, (NROWS, 128), 1)
    gid = rowi * 128 + coli
    valid = gid < N
    min_d0 = jnp.where(valid, jnp.float32(jnp.inf), jnp.float32(-1.0))

    a_row = jax.lax.broadcasted_iota(jnp.int32, (8, 128), 0)
    a_col = jax.lax.broadcasted_iota(jnp.int32, (8, 128), 1)
    a_gid = a_row * 128 + a_col
    zi = jnp.zeros((8, 128), jnp.int32)
    zf = jnp.zeros((8, 128), jnp.float32)

    def step(i, carry):
        min_d, last, idxa, cxa, cya, cza = carry
        sel = gid == last
        cx = jnp.sum(jnp.where(sel, xs, 0.0))
        cy = jnp.sum(jnp.where(sel, ys, 0.0))
        cz = jnp.sum(jnp.where(sel, zs, 0.0))
        rec = a_gid == i
        idxa = jnp.where(rec, last, idxa)
        cxa = jnp.where(rec, cx, cxa)
        cya = jnp.where(rec, cy, cya)
        cza = jnp.where(rec, cz, cza)
        dx = xs - cx
        dy = ys - cy
        dz = zs - cz
        d = dx * dx + dy * dy + dz * dz
        min_d = jnp.minimum(min_d, d)
        m = jnp.max(min_d)
        nxt = jnp.min(jnp.where(min_d == m, gid, jnp.int32(2**30)))
        return (min_d, nxt, idxa, cxa, cya, cza)

    carry = (min_d0, jnp.int32(0), zi, zf, zf, zf)
    _, _, idxa, cxa, cya, cza = jax.lax.fori_loop(0, K, step, carry)
    idx_ref[...] = idxa
    cx_ref[...] = cxa
    cy_ref[...] = cya
    cz_ref[...] = cza


def _fps(xs, ys, zs):
    out_shape = [
        jax.ShapeDtypeStruct((8, 128), jnp.int32),
        jax.ShapeDtypeStruct((8, 128), jnp.float32),
        jax.ShapeDtypeStruct((8, 128), jnp.float32),
        jax.ShapeDtypeStruct((8, 128), jnp.float32),
    ]
    return pl.pallas_call(_fps_body, out_shape=out_shape)(xs, ys, zs)


def _mlp_body(coords_ref, feats_ref, csm_ref, cv_ref,
              w1c_ref, w1f_ref, b1_ref, w2_ref, b2_ref, w3_ref, b3_ref,
              out_ref, s_scr, mf_scr):
    k = pl.program_id(0)

    @pl.when(k == 0)
    def _init():
        xc = coords_ref[:, 0:1]
        yc = coords_ref[:, 1:2]
        zc = coords_ref[:, 2:3]
        dxg = xc - cv_ref[0:1, :]
        dyg = yc - cv_ref[1:2, :]
        dzg = zc - cv_ref[2:3, :]
        dg = dxg * dxg + dyg * dyg + dzg * dzg
        mf_scr[...] = (dg < R2).astype(jnp.float32)

        w1f = w1f_ref[...]
        for i in range(NRB):
            sl = pl.ds(i * RB, RB)
            fb = feats_ref[sl, :]
            sb = jnp.dot(fb, w1f, preferred_element_type=jnp.float32)
            sb = sb + coords_ref[sl, 0:1] * w1c_ref[0:1, :]
            sb = sb + coords_ref[sl, 1:2] * w1c_ref[1:2, :]
            sb = sb + coords_ref[sl, 2:3] * w1c_ref[2:3, :]
            s_scr[sl, :] = sb + b1_ref[...]
        out_ref[...] = jnp.zeros((KPAD, L3), jnp.float32)

    cx = csm_ref[0, k]
    cy = csm_ref[1, k]
    cz = csm_ref[2, k]
    t = (cx * w1c_ref[0:1, :] + cy * w1c_ref[1:2, :] + cz * w1c_ref[2:3, :])

    lane = jax.lax.broadcasted_iota(jnp.int32, (KPAD, 1), 0)
    ek = (lane == k).astype(jnp.float32)

    w2 = w2_ref[...]
    b2 = b2_ref[...]
    w3 = w3_ref[...]
    b3 = b3_ref[...]

    def blk(i, acc):
        sl = pl.ds(i * RB, RB)
        h1 = jnp.maximum(s_scr[sl, :] - t, 0.0)
        h2 = jnp.maximum(jnp.dot(h1, w2, preferred_element_type=jnp.float32) + b2, 0.0)
        h3 = jnp.maximum(jnp.dot(h2, w3, preferred_element_type=jnp.float32) + b3, 0.0)
        cb = jnp.dot(mf_scr[sl, :], ek, preferred_element_type=jnp.float32)
        hm = jnp.where(cb > 0.5, h3, -1.0e30)
        return jnp.maximum(acc, jnp.max(hm, axis=0, keepdims=True))

    acc0 = jnp.full((1, L3), -1.0e30, jnp.float32)
    acc = jax.lax.fori_loop(0, NRB, blk, acc0)

    rowi = jax.lax.broadcasted_iota(jnp.int32, (KPAD, L3), 0)
    out_ref[...] = jnp.where(rowi == k, acc, out_ref[...])


def _mlp(coords3, feats, csm, cv, w1c, w1f, b1, w2, b2, w3, b3):
    full = lambda s: pl.BlockSpec(s, lambda k: tuple(0 for _ in s))
    in_specs = [
        full((NPAD, 3)),
        full((NPAD, 128)),
        pl.BlockSpec(memory_space=pltpu.SMEM),
        full((3, KPAD)),
        full((3, L1)),
        full((128, L1)),
        full((1, L1)),
        full((L1, L2)),
        full((1, L2)),
        full((L2, L3)),
        full((1, L3)),
    ]
    return pl.pallas_call(
        _mlp_body,
        grid=(K,),
        in_specs=in_specs,
        out_specs=full((KPAD, L3)),
        out_shape=jax.ShapeDtypeStruct((KPAD, L3), jnp.float32),
        scratch_shapes=[
            pltpu.VMEM((NPAD, L1), jnp.float32),
            pltpu.VMEM((NPAD, KPAD), jnp.float32),
        ],
        compiler_params=pltpu.CompilerParams(
            dimension_semantics=("arbitrary",),
        ),
    )(coords3, feats, csm, cv, w1c, w1f, b1, w2, b2, w3, b3)


def kernel(coords, features, W1, b1, W2, b2, W3, b3):
    xs = jnp.pad(coords[:, 0], (0, NPAD - N), constant_values=PADVAL).reshape(NROWS, 128)
    ys = jnp.pad(coords[:, 1], (0, NPAD - N), constant_values=PADVAL).reshape(NROWS, 128)
    zs = jnp.pad(coords[:, 2], (0, NPAD - N), constant_values=PADVAL).reshape(NROWS, 128)

    _, cxa, cya, cza = _fps(xs, ys, zs)
    cxf = cxa.reshape(-1)[:KPAD]
    cyf = cya.reshape(-1)[:KPAD]
    czf = cza.reshape(-1)[:KPAD]
    centers = jnp.stack([cxf[:K], cyf[:K], czf[:K]], axis=1)

    cv = jnp.stack([cxf, cyf, czf], axis=0)
    csm = cv

    coords3 = jnp.pad(coords, ((0, NPAD - N), (0, 0)), constant_values=PADVAL)
    feats = jnp.pad(features, ((0, NPAD - N), (0, 0)))

    w1c = W1[:3, :]
    w1f = W1[3:, :]
    out = _mlp(coords3, feats, csm, cv, w1c, w1f,
               b1.reshape(1, L1), W2, b2.reshape(1, L2), W3, b3.reshape(1, L3))
    return centers, out[:K, :]


# speedup vs baseline: 3.6556x; 1.7729x over previous
import jax
import jax.numpy as jnp
from jax.experimental import pallas as pl
from jax.experimental.pallas import tpu as pltpu

N = 10000
NPAD = 10240
NROWS = NPAD // 128
K = 50
KPAD = 64
R2 = 0.0625
L1, L2, L3 = 32, 32, 64
RB = 1024
NRB = NPAD // RB
PADVAL = 1.0e6


def _fps_body(xs_ref, ys_ref, zs_ref, xsm_ref, ysm_ref, zsm_ref,
              idx_ref, cx_ref, cy_ref, cz_ref):
    xs = xs_ref[...]
    ys = ys_ref[...]
    zs = zs_ref[...]
    rowi = jax.lax.broadcasted_iota(jnp.int32, (NROWS, 128), 0)
    coli = jax.lax.broadcasted_iota(jnp.int32, (NROWS, 128), 1)
    gid = rowi * 128 + coli
    valid = gid < N
    min_d0 = jnp.where(valid, jnp.float32(jnp.inf), jnp.float32(-1.0))

    a_row = jax.lax.broadcasted_iota(jnp.int32, (8, 128), 0)
    a_col = jax.lax.broadcasted_iota(jnp.int32, (8, 128), 1)
    a_gid = a_row * 128 + a_col
    zi = jnp.zeros((8, 128), jnp.int32)
    zf = jnp.zeros((8, 128), jnp.float32)

    def step(i, carry):
        min_d, last, idxa, cxa, cya, cza = carry
        cx = xsm_ref[last]
        cy = ysm_ref[last]
        cz = zsm_ref[last]
        rec = a_gid == i
        idxa = jnp.where(rec, last, idxa)
        cxa = jnp.where(rec, cx, cxa)
        cya = jnp.where(rec, cy, cya)
        cza = jnp.where(rec, cz, cza)
        dx = xs - cx
        dy = ys - cy
        dz = zs - cz
        d = dx * dx + dy * dy + dz * dz
        min_d = jnp.minimum(min_d, d)
        m = jnp.max(min_d)
        nxt = jnp.min(jnp.where(min_d == m, gid, jnp.int32(2**30)))
        return (min_d, nxt, idxa, cxa, cya, cza)

    carry = (min_d0, jnp.int32(0), zi, zf, zf, zf)
    _, _, idxa, cxa, cya, cza = jax.lax.fori_loop(0, K, step, carry)
    idx_ref[...] = idxa
    cx_ref[...] = cxa
    cy_ref[...] = cya
    cz_ref[...] = cza


def _fps(xs, ys, zs, xsf, ysf, zsf):
    out_shape = [
        jax.ShapeDtypeStruct((8, 128), jnp.int32),
        jax.ShapeDtypeStruct((8, 128), jnp.float32),
        jax.ShapeDtypeStruct((8, 128), jnp.float32),
        jax.ShapeDtypeStruct((8, 128), jnp.float32),
    ]
    full = lambda s: pl.BlockSpec(s, lambda: tuple(0 for _ in s))
    smem = pl.BlockSpec(memory_space=pltpu.SMEM)
    return pl.pallas_call(
        _fps_body,
        in_specs=[full((NROWS, 128))] * 3 + [smem] * 3,
        out_specs=[full((8, 128))] * 4,
        out_shape=out_shape,
    )(xs, ys, zs, xsf, ysf, zsf)


def _mlp_body(coords_ref, feats_ref, csm_ref, cv_ref,
              w1c_ref, w1f_ref, b1_ref, w2_ref, b2_ref, w3_ref, b3_ref,
              out_ref, s_scr, mf_scr):
    k = pl.program_id(0)

    @pl.when(k == 0)
    def _init():
        xc = coords_ref[:, 0:1]
        yc = coords_ref[:, 1:2]
        zc = coords_ref[:, 2:3]
        dxg = xc - cv_ref[0:1, :]
        dyg = yc - cv_ref[1:2, :]
        dzg = zc - cv_ref[2:3, :]
        dg = dxg * dxg + dyg * dyg + dzg * dzg
        mf_scr[...] = (dg < R2).astype(jnp.float32)

        w1f = w1f_ref[...]
        for i in range(NRB):
            sl = pl.ds(i * RB, RB)
            fb = feats_ref[sl, :]
            sb = jnp.dot(fb, w1f, preferred_element_type=jnp.float32)
            sb = sb + coords_ref[sl, 0:1] * w1c_ref[0:1, :]
            sb = sb + coords_ref[sl, 1:2] * w1c_ref[1:2, :]
            sb = sb + coords_ref[sl, 2:3] * w1c_ref[2:3, :]
            s_scr[sl, :] = sb + b1_ref[...]
        out_ref[...] = jnp.zeros((KPAD, L3), jnp.float32)

    ka = k
    kb = k + (K // 2)
    cxa = csm_ref[0, ka]
    cya = csm_ref[1, ka]
    cza = csm_ref[2, ka]
    cxb = csm_ref[0, kb]
    cyb = csm_ref[1, kb]
    czb = csm_ref[2, kb]
    ta = (cxa * w1c_ref[0:1, :] + cya * w1c_ref[1:2, :] + cza * w1c_ref[2:3, :])
    tb = (cxb * w1c_ref[0:1, :] + cyb * w1c_ref[1:2, :] + czb * w1c_ref[2:3, :])

    lane = jax.lax.broadcasted_iota(jnp.int32, (KPAD, 2), 0)
    two = jax.lax.broadcasted_iota(jnp.int32, (KPAD, 2), 1)
    ek2 = jnp.where(two == 0, (lane == ka).astype(jnp.float32),
                    (lane == kb).astype(jnp.float32))

    w2 = w2_ref[...]
    b2 = b2_ref[...]
    w3 = w3_ref[...]
    b3 = b3_ref[...]

    acc_a = jnp.full((1, L3), 0.0, jnp.float32)
    acc_b = jnp.full((1, L3), 0.0, jnp.float32)
    for i in range(NRB):
        sl = pl.ds(i * RB, RB)
        sb = s_scr[sl, :]
        cb2 = jnp.dot(mf_scr[sl, :], ek2, preferred_element_type=jnp.float32)
        h1a = jnp.maximum(sb - ta, 0.0)
        h2a = jnp.maximum(jnp.dot(h1a, w2, preferred_element_type=jnp.float32) + b2, 0.0)
        h3a = jnp.maximum(jnp.dot(h2a, w3, preferred_element_type=jnp.float32) + b3, 0.0)
        h1b = jnp.maximum(sb - tb, 0.0)
        h2b = jnp.maximum(jnp.dot(h1b, w2, preferred_element_type=jnp.float32) + b2, 0.0)
        h3b = jnp.maximum(jnp.dot(h2b, w3, preferred_element_type=jnp.float32) + b3, 0.0)
        ma = h3a * cb2[:, 0:1]
        mb = h3b * cb2[:, 1:2]
        acc_a = jnp.maximum(acc_a, jnp.max(ma, axis=0, keepdims=True))
        acc_b = jnp.maximum(acc_b, jnp.max(mb, axis=0, keepdims=True))

    rowi = jax.lax.broadcasted_iota(jnp.int32, (KPAD, L3), 0)
    o = jnp.where(rowi == ka, acc_a, out_ref[...])
    out_ref[...] = jnp.where(rowi == kb, acc_b, o)


def _mlp(coords3, feats, csm, cv, w1c, w1f, b1, w2, b2, w3, b3):
    full = lambda s: pl.BlockSpec(s, lambda k: tuple(0 for _ in s))
    in_specs = [
        full((NPAD, 3)),
        full((NPAD, 128)),
        pl.BlockSpec(memory_space=pltpu.SMEM),
        full((3, KPAD)),
        full((3, L1)),
        full((128, L1)),
        full((1, L1)),
        full((L1, L2)),
        full((1, L2)),
        full((L2, L3)),
        full((1, L3)),
    ]
    return pl.pallas_call(
        _mlp_body,
        grid=(K // 2,),
        in_specs=in_specs,
        out_specs=full((KPAD, L3)),
        out_shape=jax.ShapeDtypeStruct((KPAD, L3), jnp.float32),
        scratch_shapes=[
            pltpu.VMEM((NPAD, L1), jnp.float32),
            pltpu.VMEM((NPAD, KPAD), jnp.float32),
        ],
        compiler_params=pltpu.CompilerParams(
            dimension_semantics=("arbitrary",),
        ),
    )(coords3, feats, csm, cv, w1c, w1f, b1, w2, b2, w3, b3)


def kernel(coords, features, W1, b1, W2, b2, W3, b3):
    xs = jnp.pad(coords[:, 0], (0, NPAD - N), constant_values=PADVAL).reshape(NROWS, 128)
    ys = jnp.pad(coords[:, 1], (0, NPAD - N), constant_values=PADVAL).reshape(NROWS, 128)
    zs = jnp.pad(coords[:, 2], (0, NPAD - N), constant_values=PADVAL).reshape(NROWS, 128)

    xsf = jnp.pad(coords[:, 0], (0, NPAD - N), constant_values=PADVAL)
    ysf = jnp.pad(coords[:, 1], (0, NPAD - N), constant_values=PADVAL)
    zsf = jnp.pad(coords[:, 2], (0, NPAD - N), constant_values=PADVAL)
    _, cxa, cya, cza = _fps(xs, ys, zs, xsf, ysf, zsf)
    cxf = cxa.reshape(-1)[:KPAD]
    cyf = cya.reshape(-1)[:KPAD]
    czf = cza.reshape(-1)[:KPAD]
    centers = jnp.stack([cxf[:K], cyf[:K], czf[:K]], axis=1)

    cv = jnp.stack([cxf, cyf, czf], axis=0)
    csm = cv

    coords3 = jnp.pad(coords, ((0, NPAD - N), (0, 0)), constant_values=PADVAL)
    feats = jnp.pad(features, ((0, NPAD - N), (0, 0)))

    w1c = W1[:3, :]
    w1f = W1[3:, :]
    out = _mlp(coords3, feats, csm, cv, w1c, w1f,
               b1.reshape(1, L1), W2, b2.reshape(1, L2), W3, b3.reshape(1, L3))
    return centers, out[:K, :]


# speedup vs baseline: 3.7132x; 1.0158x over previous
import jax
import jax.numpy as jnp
from jax.experimental import pallas as pl
from jax.experimental.pallas import tpu as pltpu

N = 10000
NPAD = 10240
NROWS = NPAD // 128
K = 50
KPAD = 64
R2 = 0.0625
L1, L2, L3 = 32, 32, 64
RB = 1024
NRB = NPAD // RB
PADVAL = 1.0e6


def _fps_body(xs_ref, ys_ref, zs_ref, xsm_ref, ysm_ref, zsm_ref,
              idx_ref, cx_ref, cy_ref, cz_ref):
    xs = xs_ref[...]
    ys = ys_ref[...]
    zs = zs_ref[...]
    rowi = jax.lax.broadcasted_iota(jnp.int32, (NROWS, 128), 0)
    coli = jax.lax.broadcasted_iota(jnp.int32, (NROWS, 128), 1)
    gid = rowi * 128 + coli
    valid = gid < N
    min_d0 = jnp.where(valid, jnp.float32(jnp.inf), jnp.float32(-1.0))

    a_row = jax.lax.broadcasted_iota(jnp.int32, (8, 128), 0)
    a_col = jax.lax.broadcasted_iota(jnp.int32, (8, 128), 1)
    a_gid = a_row * 128 + a_col
    zi = jnp.zeros((8, 128), jnp.int32)
    zf = jnp.zeros((8, 128), jnp.float32)

    def step(i, carry):
        min_d, last, idxa, cxa, cya, cza = carry
        cx = xsm_ref[last]
        cy = ysm_ref[last]
        cz = zsm_ref[last]
        rec = a_gid == i
        idxa = jnp.where(rec, last, idxa)
        cxa = jnp.where(rec, cx, cxa)
        cya = jnp.where(rec, cy, cya)
        cza = jnp.where(rec, cz, cza)
        dx = xs - cx
        dy = ys - cy
        dz = zs - cz
        d = dx * dx + dy * dy + dz * dz
        min_d = jnp.minimum(min_d, d)
        m = jnp.max(min_d)
        nxt = jnp.min(jnp.where(min_d == m, gid, jnp.int32(2**30)))
        return (min_d, nxt, idxa, cxa, cya, cza)

    carry = (min_d0, jnp.int32(0), zi, zf, zf, zf)
    _, _, idxa, cxa, cya, cza = jax.lax.fori_loop(0, K, step, carry)
    idx_ref[...] = idxa
    cx_ref[...] = cxa
    cy_ref[...] = cya
    cz_ref[...] = cza


def _fps(xs, ys, zs, xsf, ysf, zsf):
    out_shape = [
        jax.ShapeDtypeStruct((8, 128), jnp.int32),
        jax.ShapeDtypeStruct((8, 128), jnp.float32),
        jax.ShapeDtypeStruct((8, 128), jnp.float32),
        jax.ShapeDtypeStruct((8, 128), jnp.float32),
    ]
    full = lambda s: pl.BlockSpec(s, lambda: tuple(0 for _ in s))
    smem = pl.BlockSpec(memory_space=pltpu.SMEM)
    return pl.pallas_call(
        _fps_body,
        in_specs=[full((NROWS, 128))] * 3 + [smem] * 3,
        out_specs=[full((8, 128))] * 4,
        out_shape=out_shape,
    )(xs, ys, zs, xsf, ysf, zsf)


NC = 4
KSTEP = 13


def _mlp_body(coords_ref, feats_ref, csm_ref, cv_ref, coordsp_ref, w1cp_ref,
              w1c_ref, w1f_ref, b1_ref, w2_ref, b2_ref, w3_ref, b3_ref,
              out_ref, s_scr, mf_scr):
    k = pl.program_id(0)

    @pl.when(k == 0)
    def _init():
        xc = coords_ref[:, 0:1]
        yc = coords_ref[:, 1:2]
        zc = coords_ref[:, 2:3]
        dxg = xc - cv_ref[0:1, :]
        dyg = yc - cv_ref[1:2, :]
        dzg = zc - cv_ref[2:3, :]
        dg = dxg * dxg + dyg * dyg + dzg * dzg
        mf_scr[...] = (dg < R2).astype(jnp.float32)

        w1f = w1f_ref[...]
        w1cp = w1cp_ref[...]
        for i in range(NRB):
            sl = pl.ds(i * RB, RB)
            sb = jnp.dot(feats_ref[sl, :], w1f, preferred_element_type=jnp.float32)
            sb = sb + jnp.dot(coordsp_ref[sl, :], w1cp, preferred_element_type=jnp.float32)
            s_scr[sl, :] = sb + b1_ref[...]
        out_ref[...] = jnp.zeros((KPAD, L3), jnp.float32)

    ks = [k + KSTEP * j for j in range(NC)]
    ts = []
    for kj in ks:
        cx = csm_ref[0, kj]
        cy = csm_ref[1, kj]
        cz = csm_ref[2, kj]
        ts.append(cx * w1c_ref[0:1, :] + cy * w1c_ref[1:2, :]
                  + cz * w1c_ref[2:3, :])

    lane = jax.lax.broadcasted_iota(jnp.int32, (KPAD, NC), 0)
    col = jax.lax.broadcasted_iota(jnp.int32, (KPAD, NC), 1)
    ek = jnp.zeros((KPAD, NC), jnp.float32)
    for j, kj in enumerate(ks):
        ek = jnp.where(jnp.logical_and(col == j, lane == kj), 1.0, ek)

    w2 = w2_ref[...]
    b2 = b2_ref[...]
    w3 = w3_ref[...]
    b3 = b3_ref[...]

    accs = [jnp.full((1, L3), 0.0, jnp.float32) for _ in range(NC)]
    for i in range(NRB):
        sl = pl.ds(i * RB, RB)
        sb = s_scr[sl, :]
        cb = jnp.dot(mf_scr[sl, :], ek, preferred_element_type=jnp.float32)
        for j in range(NC):
            h1 = jnp.maximum(sb - ts[j], 0.0)
            h2 = jnp.maximum(jnp.dot(h1, w2, preferred_element_type=jnp.float32) + b2, 0.0)
            h3 = jnp.maximum(jnp.dot(h2, w3, preferred_element_type=jnp.float32) + b3, 0.0)
            mj = h3 * cb[:, j:j + 1]
            accs[j] = jnp.maximum(accs[j], jnp.max(mj, axis=0, keepdims=True))

    rowi = jax.lax.broadcasted_iota(jnp.int32, (KPAD, L3), 0)
    o = out_ref[...]
    for j, kj in enumerate(ks):
        o = jnp.where(rowi == kj, accs[j], o)
    out_ref[...] = o


def _mlp(coords3, feats, csm, cv, coordsp, w1cp, w1c, w1f, b1, w2, b2, w3, b3):
    full = lambda s: pl.BlockSpec(s, lambda k: tuple(0 for _ in s))
    in_specs = [
        full((NPAD, 3)),
        full((NPAD, 128)),
        pl.BlockSpec(memory_space=pltpu.SMEM),
        full((3, KPAD)),
        full((NPAD, 128)),
        full((128, L1)),
        full((3, L1)),
        full((128, L1)),
        full((1, L1)),
        full((L1, L2)),
        full((1, L2)),
        full((L2, L3)),
        full((1, L3)),
    ]
    return pl.pallas_call(
        _mlp_body,
        grid=(KSTEP,),
        in_specs=in_specs,
        out_specs=full((KPAD, L3)),
        out_shape=jax.ShapeDtypeStruct((KPAD, L3), jnp.float32),
        scratch_shapes=[
            pltpu.VMEM((NPAD, L1), jnp.float32),
            pltpu.VMEM((NPAD, KPAD), jnp.float32),
        ],
        compiler_params=pltpu.CompilerParams(
            dimension_semantics=("arbitrary",),
        ),
    )(coords3, feats, csm, cv, coordsp, w1cp, w1c, w1f, b1, w2, b2, w3, b3)


def kernel(coords, features, W1, b1, W2, b2, W3, b3):
    xs = jnp.pad(coords[:, 0], (0, NPAD - N), constant_values=PADVAL).reshape(NROWS, 128)
    ys = jnp.pad(coords[:, 1], (0, NPAD - N), constant_values=PADVAL).reshape(NROWS, 128)
    zs = jnp.pad(coords[:, 2], (0, NPAD - N), constant_values=PADVAL).reshape(NROWS, 128)

    xsf = jnp.pad(coords[:, 0], (0, NPAD - N), constant_values=PADVAL)
    ysf = jnp.pad(coords[:, 1], (0, NPAD - N), constant_values=PADVAL)
    zsf = jnp.pad(coords[:, 2], (0, NPAD - N), constant_values=PADVAL)
    _, cxa, cya, cza = _fps(xs, ys, zs, xsf, ysf, zsf)
    cxf = cxa.reshape(-1)[:KPAD]
    cyf = cya.reshape(-1)[:KPAD]
    czf = cza.reshape(-1)[:KPAD]
    centers = jnp.stack([cxf[:K], cyf[:K], czf[:K]], axis=1)

    cv = jnp.stack([cxf, cyf, czf], axis=0)
    csm = cv

    coords3 = jnp.pad(coords, ((0, NPAD - N), (0, 0)), constant_values=PADVAL)
    feats = jnp.pad(features, ((0, NPAD - N), (0, 0)))

    w1c = W1[:3, :]
    w1f = W1[3:, :]
    coordsp = jnp.pad(coords3, ((0, 0), (0, 125)))
    w1cp = jnp.pad(w1c, ((0, 125), (0, 0)))
    out = _mlp(coords3, feats, csm, cv, coordsp, w1cp, w1c, w1f,
               b1.reshape(1, L1), W2, b2.reshape(1, L2), W3, b3.reshape(1, L3))
    return centers, out[:K, :]


# speedup vs baseline: 4.8725x; 1.3122x over previous
import jax
import jax.numpy as jnp
from jax.experimental import pallas as pl
from jax.experimental.pallas import tpu as pltpu

N = 10000
NPAD = 10240
NROWS = NPAD // 128
K = 50
KPAD = 64
R2 = 0.0625
L1, L2, L3 = 32, 32, 64
RB = 1024
NRB = NPAD // RB
PADVAL = 1.0e6


def _fps_body(xs_ref, ys_ref, zs_ref, xsm_ref, ysm_ref, zsm_ref,
              idx_ref, cx_ref, cy_ref, cz_ref):
    xs = xs_ref[...]
    ys = ys_ref[...]
    zs = zs_ref[...]
    rowi = jax.lax.broadcasted_iota(jnp.int32, (NROWS, 128), 0)
    coli = jax.lax.broadcasted_iota(jnp.int32, (NROWS, 128), 1)
    gid = rowi * 128 + coli
    valid = gid < N
    min_d0 = jnp.where(valid, jnp.float32(jnp.inf), jnp.float32(-1.0))

    a_row = jax.lax.broadcasted_iota(jnp.int32, (8, 128), 0)
    a_col = jax.lax.broadcasted_iota(jnp.int32, (8, 128), 1)
    a_gid = a_row * 128 + a_col
    zi = jnp.zeros((8, 128), jnp.int32)
    zf = jnp.zeros((8, 128), jnp.float32)

    def step(i, carry):
        min_d, last, idxa, cxa, cya, cza = carry
        cx = xsm_ref[last]
        cy = ysm_ref[last]
        cz = zsm_ref[last]
        rec = a_gid == i
        idxa = jnp.where(rec, last, idxa)
        cxa = jnp.where(rec, cx, cxa)
        cya = jnp.where(rec, cy, cya)
        cza = jnp.where(rec, cz, cza)
        dx = xs - cx
        dy = ys - cy
        dz = zs - cz
        d = dx * dx + dy * dy + dz * dz
        min_d = jnp.minimum(min_d, d)
        m = jnp.max(min_d)
        nxt = jnp.min(jnp.where(min_d == m, gid, jnp.int32(2**30)))
        return (min_d, nxt, idxa, cxa, cya, cza)

    carry = (min_d0, jnp.int32(0), zi, zf, zf, zf)
    _, _, idxa, cxa, cya, cza = jax.lax.fori_loop(0, K, step, carry)
    idx_ref[...] = idxa
    cx_ref[...] = cxa
    cy_ref[...] = cya
    cz_ref[...] = cza


def _fps(xs, ys, zs, xsf, ysf, zsf):
    out_shape = [
        jax.ShapeDtypeStruct((8, 128), jnp.int32),
        jax.ShapeDtypeStruct((8, 128), jnp.float32),
        jax.ShapeDtypeStruct((8, 128), jnp.float32),
        jax.ShapeDtypeStruct((8, 128), jnp.float32),
    ]
    full = lambda s: pl.BlockSpec(s, lambda: tuple(0 for _ in s))
    smem = pl.BlockSpec(memory_space=pltpu.SMEM)
    return pl.pallas_call(
        _fps_body,
        in_specs=[full((NROWS, 128))] * 3 + [smem] * 3,
        out_specs=[full((8, 128))] * 4,
        out_shape=out_shape,
    )(xs, ys, zs, xsf, ysf, zsf)


NC = 8
KSTEP = 7
LW = NC * L1
LW3 = NC * L3


def _mlp_body(coords_ref, feats_ref, csm_ref, cv_ref, coordsp_ref, w1cp_ref,
              w1ct_ref, w1f_ref, b1_ref, w2bd_ref, b2c_ref, w3bd_ref, b3c_ref,
              out_ref, s_scr, mf_scr):
    k = pl.program_id(0)

    @pl.when(k == 0)
    def _init():
        xc = coords_ref[:, 0:1]
        yc = coords_ref[:, 1:2]
        zc = coords_ref[:, 2:3]
        dxg = xc - cv_ref[0:1, :]
        dyg = yc - cv_ref[1:2, :]
        dzg = zc - cv_ref[2:3, :]
        dg = dxg * dxg + dyg * dyg + dzg * dzg
        mf_scr[...] = (dg < R2).astype(jnp.float32)

        w1f = w1f_ref[...]
        w1cp = w1cp_ref[...]
        for i in range(NRB):
            sl = pl.ds(i * RB, RB)
            sb = jnp.dot(feats_ref[sl, :], w1f, preferred_element_type=jnp.float32)
            sb = sb + jnp.dot(coordsp_ref[sl, :], w1cp, preferred_element_type=jnp.float32)
            sb = sb + b1_ref[...]
            for j in range(NC):
                s_scr[sl, pl.ds(j * L1, L1)] = sb
        out_ref[...] = jnp.zeros((KPAD, L3), jnp.float32)

    ks = [k + KSTEP * j for j in range(NC)]
    grp = jax.lax.broadcasted_iota(jnp.int32, (1, LW), 1) // L1
    cxs = jnp.zeros((1, LW), jnp.float32)
    cys = jnp.zeros((1, LW), jnp.float32)
    czs = jnp.zeros((1, LW), jnp.float32)
    for j, kj in enumerate(ks):
        cxs = jnp.where(grp == j, csm_ref[0, kj], cxs)
        cys = jnp.where(grp == j, csm_ref[1, kj], cys)
        czs = jnp.where(grp == j, csm_ref[2, kj], czs)
    t8 = (cxs * w1ct_ref[0:1, :] + cys * w1ct_ref[1:2, :]
          + czs * w1ct_ref[2:3, :])

    rowe = jax.lax.broadcasted_iota(jnp.int32, (KPAD, LW3), 0)
    cole = jax.lax.broadcasted_iota(jnp.int32, (KPAD, LW3), 1) // L3
    ee = jnp.zeros((KPAD, LW3), jnp.float32)
    for j, kj in enumerate(ks):
        ee = jnp.where(jnp.logical_and(cole == j, rowe == kj), 1.0, ee)

    w2bd = w2bd_ref[...]
    b2c = b2c_ref[...]
    w3bd = w3bd_ref[...]
    b3c = b3c_ref[...]

    acc = jnp.zeros((1, LW3), jnp.float32)
    for i in range(NRB):
        sl = pl.ds(i * RB, RB)
        h1 = jnp.maximum(s_scr[sl, :] - t8, 0.0)
        h2 = jnp.maximum(jnp.dot(h1, w2bd, preferred_element_type=jnp.float32) + b2c, 0.0)
        h3 = jnp.maximum(jnp.dot(h2, w3bd, preferred_element_type=jnp.float32) + b3c, 0.0)
        cbx = jnp.dot(mf_scr[sl, :], ee, preferred_element_type=jnp.float32)
        hm = h3 * cbx
        acc = jnp.maximum(acc, jnp.max(hm, axis=0, keepdims=True))

    rowi = jax.lax.broadcasted_iota(jnp.int32, (KPAD, L3), 0)
    o = out_ref[...]
    for j, kj in enumerate(ks):
        o = jnp.where(rowi == kj, acc[:, j * L3:(j + 1) * L3], o)
    out_ref[...] = o


def _mlp(coords3, feats, csm, cv, coordsp, w1cp, w1ct, w1f, b1,
         w2bd, b2c, w3bd, b3c):
    full = lambda s: pl.BlockSpec(s, lambda k: tuple(0 for _ in s))
    in_specs = [
        full((NPAD, 3)),
        full((NPAD, 128)),
        pl.BlockSpec(memory_space=pltpu.SMEM),
        full((3, KPAD)),
        full((NPAD, 128)),
        full((128, L1)),
        full((3, LW)),
        full((128, L1)),
        full((1, L1)),
        full((LW, LW)),
        full((1, LW)),
        full((LW, LW3)),
        full((1, LW3)),
    ]
    return pl.pallas_call(
        _mlp_body,
        grid=(KSTEP,),
        in_specs=in_specs,
        out_specs=full((KPAD, L3)),
        out_shape=jax.ShapeDtypeStruct((KPAD, L3), jnp.float32),
        scratch_shapes=[
            pltpu.VMEM((NPAD, LW), jnp.float32),
            pltpu.VMEM((NPAD, KPAD), jnp.float32),
        ],
        compiler_params=pltpu.CompilerParams(
            dimension_semantics=("arbitrary",),
        ),
    )(coords3, feats, csm, cv, coordsp, w1cp, w1ct, w1f, b1,
      w2bd, b2c, w3bd, b3c)


def kernel(coords, features, W1, b1, W2, b2, W3, b3):
    xs = jnp.pad(coords[:, 0], (0, NPAD - N), constant_values=PADVAL).reshape(NROWS, 128)
    ys = jnp.pad(coords[:, 1], (0, NPAD - N), constant_values=PADVAL).reshape(NROWS, 128)
    zs = jnp.pad(coords[:, 2], (0, NPAD - N), constant_values=PADVAL).reshape(NROWS, 128)

    xsf = jnp.pad(coords[:, 0], (0, NPAD - N), constant_values=PADVAL)
    ysf = jnp.pad(coords[:, 1], (0, NPAD - N), constant_values=PADVAL)
    zsf = jnp.pad(coords[:, 2], (0, NPAD - N), constant_values=PADVAL)
    _, cxa, cya, cza = _fps(xs, ys, zs, xsf, ysf, zsf)
    cxf = cxa.reshape(-1)[:KPAD]
    cyf = cya.reshape(-1)[:KPAD]
    czf = cza.reshape(-1)[:KPAD]
    centers = jnp.stack([cxf[:K], cyf[:K], czf[:K]], axis=1)

    cv = jnp.stack([cxf, cyf, czf], axis=0)
    csm = cv

    coords3 = jnp.pad(coords, ((0, NPAD - N), (0, 0)), constant_values=PADVAL)
    feats = jnp.pad(features, ((0, NPAD - N), (0, 0)))

    w1c = W1[:3, :]
    w1f = W1[3:, :]
    coordsp = jnp.pad(coords3, ((0, 0), (0, 125)))
    w1cp = jnp.pad(w1c, ((0, 125), (0, 0)))
    w1ct = jnp.tile(w1c, (1, NC))
    w2bd = jnp.kron(jnp.eye(NC, dtype=jnp.float32), W2)
    w3bd = jnp.kron(jnp.eye(NC, dtype=jnp.float32), W3)
    b2c = jnp.tile(b2.reshape(1, L2), (1, NC))
    b3c = jnp.tile(b3.reshape(1, L3), (1, NC))
    out = _mlp(coords3, feats, csm, cv, coordsp, w1cp, w1ct, w1f,
               b1.reshape(1, L1), w2bd, b2c, w3bd, b3c)
    return centers, out[:K, :]
